# Initial kernel scaffold; baseline (speedup 1.0000x reference)
import jax
import jax.numpy as jnp
from jax.experimental import pallas as pl


def kernel(x, edge_index, W1, b1, W2, b2, W3, b3, W4, b4):
    raise NotImplementedError("write your pallas kernel here")


# speedup vs baseline: 10.1318x; 10.1318x over previous
import functools

import jax
import jax.numpy as jnp
from jax import lax
from jax.experimental import pallas as pl
from jax.experimental.pallas import tpu as pltpu
from jax.experimental.pallas import tpu_sc as plsc

N = 10000
E = 160000
NC, NS = 2, 16
NW = NC * NS
KB = 128
NB = 40
EW = KB * NB
EP = EW * NW
NROW = 10240
RPT = NROW // NS
DUMP = N

P1, P2, P3, P4 = 112, 64, 32, 16
RB = 1000


def _deg_body(dst_hbm, deg_out, dst_v, degtab, sumv, shared):
    cid = lax.axis_index("c")
    sid = lax.axis_index("s")
    pltpu.sync_copy(dst_hbm.at[cid, sid], dst_v)

    def _zero(i, _):
        degtab[pl.ds(i * 16, 16)] = jnp.zeros((16,), jnp.float32)
        return 0
    lax.fori_loop(0, NROW // 16, _zero, 0)

    ones = jnp.ones((16,), jnp.float32)

    def _hist(i, _):
        j = i // (KB // 16)
        c = i % (KB // 16)
        idx = dst_v[j, pl.ds(c * 16, 16)]
        plsc.addupdate_scatter(degtab, [idx], ones)
        return 0
    lax.fori_loop(0, EW // 16, _hist, 0)

    pltpu.sync_copy(degtab, shared.at[sid])
    plsc.subcore_barrier()
    pltpu.sync_copy(shared.at[:, pl.ds(sid * RPT, RPT)], sumv)

    def _red(ci, _):
        a = jnp.zeros((16,), jnp.float32)
        for r in range(NS):
            a = a + sumv[r, pl.ds(ci * 16, 16)]
        degtab[pl.ds(ci * 16, 16)] = a
        return 0
    lax.fori_loop(0, RPT // 16, _red, 0)
    pltpu.sync_copy(degtab.at[pl.ds(0, RPT)], deg_out.at[cid, pl.ds(sid * RPT, RPT)])


_deg_kernel = pl.kernel(
    _deg_body,
    out_type=jax.ShapeDtypeStruct((NC, NROW), jnp.float32),
    mesh=plsc.VectorSubcoreMesh(core_axis_name="c", subcore_axis_name="s"),
    compiler_params=pltpu.CompilerParams(needs_layout_passes=False),
    scratch_types=[
        pltpu.VMEM((NB, KB), jnp.int32),
        pltpu.VMEM((NROW,), jnp.float32),
        pltpu.VMEM((NS, RPT), jnp.float32),
        pltpu.VMEM_SHARED((NS, NROW), jnp.float32),
    ],
)


def _agg_body(yw_hbm, src_hbm, dst_hbm, zeros_hbm, acc_out,
              src_v, dst_v, rows_v, gsem, acc_sp):
    cid = lax.axis_index("c")
    sid = lax.axis_index("s")
    pltpu.sync_copy(zeros_hbm.at[pl.ds(sid * RPT, RPT)],
                    acc_sp.at[pl.ds(sid * RPT, RPT)])
    plsc.subcore_barrier()
    pltpu.sync_copy(src_hbm.at[cid, sid], src_v)
    pltpu.sync_copy(dst_hbm.at[cid, sid], dst_v)

    def _batch(j, _):
        pltpu.async_copy(yw_hbm.at[src_v.at[j]], rows_v, gsem).wait()
        pltpu.sync_copy(rows_v, acc_sp.at[dst_v.at[j]], add=True)
        return 0
    lax.fori_loop(0, NB, _batch, 0)
    plsc.subcore_barrier()
    pltpu.sync_copy(acc_sp.at[pl.ds(sid * RPT, RPT)],
                    acc_out.at[cid, pl.ds(sid * RPT, RPT)])


def _make_agg(d):
    return pl.kernel(
        _agg_body,
        out_type=jax.ShapeDtypeStruct((NC, NROW, d), jnp.float32),
        mesh=plsc.VectorSubcoreMesh(core_axis_name="c", subcore_axis_name="s"),
        compiler_params=pltpu.CompilerParams(use_tc_tiling_on_sc=False),
        scratch_types=[
            pltpu.VMEM((NB, KB), jnp.int32),
            pltpu.VMEM((NB, KB), jnp.int32),
            pltpu.VMEM((KB, d), jnp.float32),
            pltpu.SemaphoreType.DMA,
            pltpu.VMEM_SHARED((NROW, d), jnp.float32),
        ],
    )


def _pre_body(x_ref, w_ref, degt_ref, yw_ref, dinv_ref):
    d = degt_ref[:, 0:1] + degt_ref[:, 1:2] + 1.0
    dv = lax.rsqrt(d)
    xw = jnp.dot(x_ref[...], w_ref[...], preferred_element_type=jnp.float32)
    yw_ref[...] = dv * xw
    dinv_ref[...] = dv


def _tc_pre(x, w1p, degt):
    return pl.pallas_call(
        _pre_body,
        grid=(N // RB,),
        in_specs=[
            pl.BlockSpec((RB, x.shape[1]), lambda i: (i, 0)),
            pl.BlockSpec((w1p.shape[0], w1p.shape[1]), lambda i: (0, 0)),
            pl.BlockSpec((RB, 2), lambda i: (i, 0)),
        ],
        out_specs=[
            pl.BlockSpec((RB, w1p.shape[1]), lambda i: (i, 0)),
            pl.BlockSpec((RB, 1), lambda i: (i, 0)),
        ],
        out_shape=[
            jax.ShapeDtypeStruct((N, w1p.shape[1]), jnp.float32),
            jax.ShapeDtypeStruct((N, 1), jnp.float32),
        ],
    )(x, w1p, degt)


def _mid_body(acca_ref, accb_ref, yw_ref, dinv_ref, b_ref, w_ref, out_ref):
    dv = dinv_ref[...]
    h = dv * (acca_ref[0] + accb_ref[0] + yw_ref[...]) + b_ref[...]
    h = jnp.maximum(h, 0.0)
    out_ref[...] = dv * jnp.dot(h, w_ref[...], preferred_element_type=jnp.float32)


def _tc_mid(acc2, yw, dinv, bp, wp):
    din, dout = wp.shape
    return pl.pallas_call(
        _mid_body,
        grid=(N // RB,),
        in_specs=[
            pl.BlockSpec((1, RB, din), lambda i: (0, i, 0)),
            pl.BlockSpec((1, RB, din), lambda i: (1, i, 0)),
            pl.BlockSpec((RB, din), lambda i: (i, 0)),
            pl.BlockSpec((RB, 1), lambda i: (i, 0)),
            pl.BlockSpec((1, din), lambda i: (0, 0)),
            pl.BlockSpec((din, dout), lambda i: (0, 0)),
        ],
        out_specs=pl.BlockSpec((RB, dout), lambda i: (i, 0)),
        out_shape=jax.ShapeDtypeStruct((N, dout), jnp.float32),
    )(acc2, acc2, yw, dinv, bp, wp)


def _post_body(acca_ref, accb_ref, yw_ref, dinv_ref, b_ref, out_ref):
    dv = dinv_ref[...]
    out_ref[...] = dv * (acca_ref[0] + accb_ref[0] + yw_ref[...]) + b_ref[...]


def _tc_post(acc2, yw, dinv, bp):
    din = yw.shape[1]
    return pl.pallas_call(
        _post_body,
        grid=(N // RB,),
        in_specs=[
            pl.BlockSpec((1, RB, din), lambda i: (0, i, 0)),
            pl.BlockSpec((1, RB, din), lambda i: (1, i, 0)),
            pl.BlockSpec((RB, din), lambda i: (i, 0)),
            pl.BlockSpec((RB, 1), lambda i: (i, 0)),
            pl.BlockSpec((1, din), lambda i: (0, 0)),
        ],
        out_specs=pl.BlockSpec((RB, din), lambda i: (i, 0)),
        out_shape=jax.ShapeDtypeStruct((N, din), jnp.float32),
    )(acc2, acc2, yw, dinv, bp)


def _pad2(a, rows, cols):
    return jnp.pad(a, ((0, rows - a.shape[0]), (0, cols - a.shape[1])))


def kernel(x, edge_index, W1, b1, W2, b2, W3, b3, W4, b4):
    f32 = jnp.float32
    src = edge_index[0].astype(jnp.int32)
    dst = edge_index[1].astype(jnp.int32)
    pad = EP - E
    src_r = jnp.concatenate([src, jnp.zeros((pad,), jnp.int32)]).reshape(NC, NS, NB, KB)
    dst_r = jnp.concatenate([dst, jnp.full((pad,), DUMP, jnp.int32)]).reshape(NC, NS, NB, KB)

    w1p = _pad2(W1, 256, P1)
    w2p = _pad2(W2, P1, P2)
    w3p = _pad2(W3, P2, P3)
    w4p = _pad2(W4, P3, P4)
    b1p = jnp.pad(b1, (0, P1 - b1.shape[0])).reshape(1, P1)
    b2p = jnp.pad(b2, (0, P2 - b2.shape[0])).reshape(1, P2)
    b3p = jnp.pad(b3, (0, P3 - b3.shape[0])).reshape(1, P3)
    b4p = jnp.pad(b4, (0, P4 - b4.shape[0])).reshape(1, P4)

    deg2 = _deg_kernel(dst_r)
    degt = deg2.T[:N]

    yw1, dinv = _tc_pre(x, w1p, degt)
    acc1 = _make_agg(P1)(yw1, src_r, dst_r, jnp.zeros((NROW, P1), f32))
    yw2 = _tc_mid(acc1, yw1, dinv, b1p, w2p)
    acc2 = _make_agg(P2)(yw2, src_r, dst_r, jnp.zeros((NROW, P2), f32))
    yw3 = _tc_mid(acc2, yw2, dinv, b2p, w3p)
    acc3 = _make_agg(P3)(yw3, src_r, dst_r, jnp.zeros((NROW, P3), f32))
    yw4 = _tc_mid(acc3, yw3, dinv, b3p, w4p)
    acc4 = _make_agg(P4)(yw4, src_r, dst_r, jnp.zeros((NROW, P4), f32))
    out = _tc_post(acc4, yw4, dinv, b4p)
    return out[:, :1]


# speedup vs baseline: 11.5726x; 1.1422x over previous
import functools

import jax
import jax.numpy as jnp
from jax import lax
from jax.experimental import pallas as pl
from jax.experimental.pallas import tpu as pltpu
from jax.experimental.pallas import tpu_sc as plsc

N = 10000
E = 160000
NC, NS = 2, 16
NW = NC * NS
KB = 128
NB = 40
EW = KB * NB
EP = EW * NW
NROW = 10240
RPT = NROW // NS
DUMP = N

P1, P2, P3, P4 = 112, 64, 32, 16
RB = 1000


def _deg_body(dst_hbm, deg_out, dst_v, degtab, sumv, shared):
    cid = lax.axis_index("c")
    sid = lax.axis_index("s")
    pltpu.sync_copy(dst_hbm.at[cid, sid], dst_v)

    def _zero(i, _):
        degtab[pl.ds(i * 16, 16)] = jnp.zeros((16,), jnp.float32)
        return 0
    lax.fori_loop(0, NROW // 16, _zero, 0)

    ones = jnp.ones((16,), jnp.float32)

    def _hist(i, _):
        j = i // (KB // 16)
        c = i % (KB // 16)
        idx = dst_v[j, pl.ds(c * 16, 16)]
        plsc.addupdate_scatter(degtab, [idx], ones)
        return 0
    lax.fori_loop(0, EW // 16, _hist, 0)

    pltpu.sync_copy(degtab, shared.at[sid])
    plsc.subcore_barrier()
    pltpu.sync_copy(shared.at[:, pl.ds(sid * RPT, RPT)], sumv)

    def _red(ci, _):
        a = jnp.zeros((16,), jnp.float32)
        for r in range(NS):
            a = a + sumv[r, pl.ds(ci * 16, 16)]
        degtab[pl.ds(ci * 16, 16)] = a
        return 0
    lax.fori_loop(0, RPT // 16, _red, 0)
    pltpu.sync_copy(degtab.at[pl.ds(0, RPT)], deg_out.at[cid, pl.ds(sid * RPT, RPT)])


_deg_kernel = pl.kernel(
    _deg_body,
    out_type=jax.ShapeDtypeStruct((NC, NROW), jnp.float32),
    mesh=plsc.VectorSubcoreMesh(core_axis_name="c", subcore_axis_name="s"),
    compiler_params=pltpu.CompilerParams(needs_layout_passes=False),
    scratch_types=[
        pltpu.VMEM((NB, KB), jnp.int32),
        pltpu.VMEM((NROW,), jnp.float32),
        pltpu.VMEM((NS, RPT), jnp.float32),
        pltpu.VMEM_SHARED((NS, NROW), jnp.float32),
    ],
)


def _make_agg_body(nbuf, kb):
    nb = EW // kb

    def _agg_body(yw_hbm, src_hbm, dst_hbm, zeros_hbm, acc_out,
                  src_v, dst_v, *rest):
        rows = rest[:nbuf]
        gsems = rest[nbuf:2 * nbuf]
        ssems = rest[2 * nbuf:3 * nbuf]
        acc_sp = rest[3 * nbuf]
        cid = lax.axis_index("c")
        sid = lax.axis_index("s")
        pltpu.sync_copy(zeros_hbm.at[pl.ds(sid * RPT, RPT)],
                        acc_sp.at[pl.ds(sid * RPT, RPT)])
        plsc.subcore_barrier()
        pltpu.sync_copy(src_hbm.at[cid, sid], src_v)
        pltpu.sync_copy(dst_hbm.at[cid, sid], dst_v)

        def _gather(i, b, sem):
            return pltpu.make_async_copy(yw_hbm.at[src_v.at[i]], rows[b], sem)

        def _scatter(i, b, sem):
            return pltpu.make_async_copy(rows[b], acc_sp.at[dst_v.at[i]], sem)

        for b in range(nbuf):
            _gather(b, b, gsems[b]).start()

        nw = nb // nbuf

        def _wave(w, _):
            i0 = w * nbuf
            for b in range(nbuf):
                _gather(i0 + b, b, gsems[b]).wait()
                _scatter(i0 + b, b, ssems[b]).start(add=True)
            for b in range(nbuf):
                _scatter(i0 + b, b, ssems[b]).wait()
                _gather(i0 + nbuf + b, b, gsems[b]).start()
            return 0
        lax.fori_loop(0, nw - 1, _wave, 0)
        i0 = (nw - 1) * nbuf
        for b in range(nbuf):
            _gather(i0 + b, b, gsems[b]).wait()
            _scatter(i0 + b, b, ssems[b]).start(add=True)
        for b in range(nbuf):
            _scatter(i0 + b, b, ssems[b]).wait()
        plsc.subcore_barrier()
        pltpu.sync_copy(acc_sp.at[pl.ds(sid * RPT, RPT)],
                        acc_out.at[cid, pl.ds(sid * RPT, RPT)])
    return _agg_body


def _make_agg(d, nbuf, kb):
    nb = EW // kb
    return pl.kernel(
        _make_agg_body(nbuf, kb),
        out_type=jax.ShapeDtypeStruct((NC, NROW, d), jnp.float32),
        mesh=plsc.VectorSubcoreMesh(core_axis_name="c", subcore_axis_name="s"),
        compiler_params=pltpu.CompilerParams(use_tc_tiling_on_sc=False),
        scratch_types=(
            [pltpu.VMEM((nb, kb), jnp.int32),
             pltpu.VMEM((nb, kb), jnp.int32)]
            + [pltpu.VMEM((kb, d), jnp.float32) for _ in range(nbuf)]
            + [pltpu.SemaphoreType.DMA for _ in range(2 * nbuf)]
            + [pltpu.VMEM_SHARED((NROW, d), jnp.float32)]
        ),
    )


def _pre_body(x_ref, w_ref, degt_ref, yw_ref, dinv_ref):
    d = degt_ref[:, 0:1] + degt_ref[:, 1:2] + 1.0
    dv = lax.rsqrt(d)
    xw = jnp.dot(x_ref[...], w_ref[...], preferred_element_type=jnp.float32)
    yw_ref[...] = dv * xw
    dinv_ref[...] = dv


def _tc_pre(x, w1p, degt):
    return pl.pallas_call(
        _pre_body,
        grid=(N // RB,),
        in_specs=[
            pl.BlockSpec((RB, x.shape[1]), lambda i: (i, 0)),
            pl.BlockSpec((w1p.shape[0], w1p.shape[1]), lambda i: (0, 0)),
            pl.BlockSpec((RB, 2), lambda i: (i, 0)),
        ],
        out_specs=[
            pl.BlockSpec((RB, w1p.shape[1]), lambda i: (i, 0)),
            pl.BlockSpec((RB, 1), lambda i: (i, 0)),
        ],
        out_shape=[
            jax.ShapeDtypeStruct((N, w1p.shape[1]), jnp.float32),
            jax.ShapeDtypeStruct((N, 1), jnp.float32),
        ],
    )(x, w1p, degt)


def _mid_body(acca_ref, accb_ref, yw_ref, dinv_ref, b_ref, w_ref, out_ref):
    dv = dinv_ref[...]
    h = dv * (acca_ref[0] + accb_ref[0] + yw_ref[...]) + b_ref[...]
    h = jnp.maximum(h, 0.0)
    out_ref[...] = dv * jnp.dot(h, w_ref[...], preferred_element_type=jnp.float32)


def _tc_mid(acc2, yw, dinv, bp, wp):
    din, dout = wp.shape
    return pl.pallas_call(
        _mid_body,
        grid=(N // RB,),
        in_specs=[
            pl.BlockSpec((1, RB, din), lambda i: (0, i, 0)),
            pl.BlockSpec((1, RB, din), lambda i: (1, i, 0)),
            pl.BlockSpec((RB, din), lambda i: (i, 0)),
            pl.BlockSpec((RB, 1), lambda i: (i, 0)),
            pl.BlockSpec((1, din), lambda i: (0, 0)),
            pl.BlockSpec((din, dout), lambda i: (0, 0)),
        ],
        out_specs=pl.BlockSpec((RB, dout), lambda i: (i, 0)),
        out_shape=jax.ShapeDtypeStruct((N, dout), jnp.float32),
    )(acc2, acc2, yw, dinv, bp, wp)


def _post_body(acca_ref, accb_ref, yw_ref, dinv_ref, b_ref, out_ref):
    dv = dinv_ref[...]
    out_ref[...] = dv * (acca_ref[0] + accb_ref[0] + yw_ref[...]) + b_ref[...]


def _tc_post(acc2, yw, dinv, bp):
    din = yw.shape[1]
    return pl.pallas_call(
        _post_body,
        grid=(N // RB,),
        in_specs=[
            pl.BlockSpec((1, RB, din), lambda i: (0, i, 0)),
            pl.BlockSpec((1, RB, din), lambda i: (1, i, 0)),
            pl.BlockSpec((RB, din), lambda i: (i, 0)),
            pl.BlockSpec((RB, 1), lambda i: (i, 0)),
            pl.BlockSpec((1, din), lambda i: (0, 0)),
        ],
        out_specs=pl.BlockSpec((RB, din), lambda i: (i, 0)),
        out_shape=jax.ShapeDtypeStruct((N, din), jnp.float32),
    )(acc2, acc2, yw, dinv, bp)


def _pad2(a, rows, cols):
    return jnp.pad(a, ((0, rows - a.shape[0]), (0, cols - a.shape[1])))


def kernel(x, edge_index, W1, b1, W2, b2, W3, b3, W4, b4):
    f32 = jnp.float32
    src = edge_index[0].astype(jnp.int32)
    dst = edge_index[1].astype(jnp.int32)
    pad = EP - E
    src_r = jnp.concatenate([src, jnp.zeros((pad,), jnp.int32)]).reshape(NC, NS, NB, KB)
    dst_r = jnp.concatenate([dst, jnp.full((pad,), DUMP, jnp.int32)]).reshape(NC, NS, NB, KB)

    w1p = _pad2(W1, 256, P1)
    w2p = _pad2(W2, P1, P2)
    w3p = _pad2(W3, P2, P3)
    w4p = _pad2(W4, P3, P4)
    b1p = jnp.pad(b1, (0, P1 - b1.shape[0])).reshape(1, P1)
    b2p = jnp.pad(b2, (0, P2 - b2.shape[0])).reshape(1, P2)
    b3p = jnp.pad(b3, (0, P3 - b3.shape[0])).reshape(1, P3)
    b4p = jnp.pad(b4, (0, P4 - b4.shape[0])).reshape(1, P4)

    deg2 = _deg_kernel(dst_r)
    degt = deg2.T[:N]

    yw1, dinv = _tc_pre(x, w1p, degt)
    src_r64 = src_r.reshape(NC, NS, EW // 64, 64)
    dst_r64 = dst_r.reshape(NC, NS, EW // 64, 64)
    acc1 = _make_agg(P1, 4, 64)(yw1, src_r64, dst_r64, jnp.zeros((NROW, P1), f32))
    yw2 = _tc_mid(acc1, yw1, dinv, b1p, w2p)
    acc2 = _make_agg(P2, 8, 128)(yw2, src_r, dst_r, jnp.zeros((NROW, P2), f32))
    yw3 = _tc_mid(acc2, yw2, dinv, b2p, w3p)
    acc3 = _make_agg(P3, 8, 128)(yw3, src_r, dst_r, jnp.zeros((NROW, P3), f32))
    yw4 = _tc_mid(acc3, yw3, dinv, b3p, w4p)
    acc4 = _make_agg(P4, 8, 128)(yw4, src_r, dst_r, jnp.zeros((NROW, P4), f32))
    out = _tc_post(acc4, yw4, dinv, b4p)
    return out[:, :1]


# speedup vs baseline: 12.1437x; 1.0494x over previous
import functools

import jax
import jax.numpy as jnp
from jax import lax
from jax.experimental import pallas as pl
from jax.experimental.pallas import tpu as pltpu
from jax.experimental.pallas import tpu_sc as plsc

N = 10000
E = 160000
NC, NS = 2, 16
NW = NC * NS
KB = 128
NB = 40
EW = KB * NB
EP = EW * NW
NROW = 10240
RPT = NROW // NS
DUMP = N

P1, P2, P3, P4 = 112, 64, 32, 16
RB = 1000


def _deg_body(dst_hbm, deg_out, dst_v, degtab, sumv, shared):
    cid = lax.axis_index("c")
    sid = lax.axis_index("s")
    pltpu.sync_copy(dst_hbm.at[cid, sid], dst_v)

    def _zero(i, _):
        degtab[pl.ds(i * 16, 16)] = jnp.zeros((16,), jnp.float32)
        return 0
    lax.fori_loop(0, NROW // 16, _zero, 0)

    ones = jnp.ones((16,), jnp.float32)

    def _hist(i, _):
        j = i // (KB // 16)
        c = i % (KB // 16)
        idx = dst_v[j, pl.ds(c * 16, 16)]
        plsc.addupdate_scatter(degtab, [idx], ones)
        return 0
    lax.fori_loop(0, EW // 16, _hist, 0)

    pltpu.sync_copy(degtab, shared.at[sid])
    plsc.subcore_barrier()
    pltpu.sync_copy(shared.at[:, pl.ds(sid * RPT, RPT)], sumv)

    def _red(ci, _):
        a = jnp.zeros((16,), jnp.float32)
        for r in range(NS):
            a = a + sumv[r, pl.ds(ci * 16, 16)]
        degtab[pl.ds(ci * 16, 16)] = a
        return 0
    lax.fori_loop(0, RPT // 16, _red, 0)
    pltpu.sync_copy(degtab.at[pl.ds(0, RPT)], deg_out.at[cid, pl.ds(sid * RPT, RPT)])


_deg_kernel = pl.kernel(
    _deg_body,
    out_type=jax.ShapeDtypeStruct((NC, NROW), jnp.float32),
    mesh=plsc.VectorSubcoreMesh(core_axis_name="c", subcore_axis_name="s"),
    compiler_params=pltpu.CompilerParams(needs_layout_passes=False),
    scratch_types=[
        pltpu.VMEM((NB, KB), jnp.int32),
        pltpu.VMEM((NROW,), jnp.float32),
        pltpu.VMEM((NS, RPT), jnp.float32),
        pltpu.VMEM_SHARED((NS, NROW), jnp.float32),
    ],
)


def _make_agg_body(nbuf, kb, nb0, nb1):
    def _agg_body(yw_hbm, src_hbm, dst_hbm, zeros_hbm, acc_out,
                  src_v, dst_v, *rest):
        rows = rest[:nbuf]
        gsems = rest[nbuf:2 * nbuf]
        ssems = rest[2 * nbuf:3 * nbuf]
        acc_sp = rest[3 * nbuf]
        cid = lax.axis_index("c")
        sid = lax.axis_index("s")
        pltpu.sync_copy(zeros_hbm.at[pl.ds(sid * RPT, RPT)],
                        acc_sp.at[pl.ds(sid * RPT, RPT)])
        plsc.subcore_barrier()
        pltpu.sync_copy(src_hbm.at[cid, sid], src_v)
        pltpu.sync_copy(dst_hbm.at[cid, sid], dst_v)

        def _gather(i, b, sem):
            return pltpu.make_async_copy(yw_hbm.at[src_v.at[i]], rows[b], sem)

        def _scatter(i, b, sem):
            return pltpu.make_async_copy(rows[b], acc_sp.at[dst_v.at[i]], sem)

        for b in range(nbuf):
            _gather(b, b, gsems[b]).start()

        nw = jnp.where(cid == 0, nb0 // nbuf, nb1 // nbuf)

        def _wave(w, _):
            i0 = w * nbuf
            for b in range(nbuf):
                _gather(i0 + b, b, gsems[b]).wait()
                _scatter(i0 + b, b, ssems[b]).start(add=True)
            for b in range(nbuf):
                _scatter(i0 + b, b, ssems[b]).wait()
                _gather(i0 + nbuf + b, b, gsems[b]).start()
            return 0
        lax.fori_loop(0, nw - 1, _wave, 0)
        i0 = (nw - 1) * nbuf
        for b in range(nbuf):
            _gather(i0 + b, b, gsems[b]).wait()
            _scatter(i0 + b, b, ssems[b]).start(add=True)
        for b in range(nbuf):
            _scatter(i0 + b, b, ssems[b]).wait()
        plsc.subcore_barrier()
        pltpu.sync_copy(acc_sp.at[pl.ds(sid * RPT, RPT)],
                        acc_out.at[cid, pl.ds(sid * RPT, RPT)])
    return _agg_body


def _make_agg(d, nbuf, kb, nb0, nb1):
    nb = max(nb0, nb1)
    return pl.kernel(
        _make_agg_body(nbuf, kb, nb0, nb1),
        out_type=jax.ShapeDtypeStruct((NC, NROW, d), jnp.float32),
        mesh=plsc.VectorSubcoreMesh(core_axis_name="c", subcore_axis_name="s"),
        compiler_params=pltpu.CompilerParams(use_tc_tiling_on_sc=False),
        scratch_types=(
            [pltpu.VMEM((nb, kb), jnp.int32),
             pltpu.VMEM((nb, kb), jnp.int32)]
            + [pltpu.VMEM((kb, d), jnp.float32) for _ in range(nbuf)]
            + [pltpu.SemaphoreType.DMA for _ in range(2 * nbuf)]
            + [pltpu.VMEM_SHARED((NROW, d), jnp.float32)]
        ),
    )


def _pre_body(x_ref, w_ref, degt_ref, yw_ref, dinv_ref):
    d = degt_ref[:, 0:1] + degt_ref[:, 1:2] + 1.0
    dv = lax.rsqrt(d)
    xw = jnp.dot(x_ref[...], w_ref[...], preferred_element_type=jnp.float32)
    yw_ref[...] = dv * xw
    dinv_ref[...] = dv


def _tc_pre(x, w1p, degt):
    return pl.pallas_call(
        _pre_body,
        grid=(N // RB,),
        in_specs=[
            pl.BlockSpec((RB, x.shape[1]), lambda i: (i, 0)),
            pl.BlockSpec((w1p.shape[0], w1p.shape[1]), lambda i: (0, 0)),
            pl.BlockSpec((RB, 2), lambda i: (i, 0)),
        ],
        out_specs=[
            pl.BlockSpec((RB, w1p.shape[1]), lambda i: (i, 0)),
            pl.BlockSpec((RB, 1), lambda i: (i, 0)),
        ],
        out_shape=[
            jax.ShapeDtypeStruct((N, w1p.shape[1]), jnp.float32),
            jax.ShapeDtypeStruct((N, 1), jnp.float32),
        ],
    )(x, w1p, degt)


def _mid_body(acca_ref, accb_ref, yw_ref, dinv_ref, b_ref, w_ref, out_ref):
    dv = dinv_ref[...]
    h = dv * (acca_ref[0] + accb_ref[0] + yw_ref[...]) + b_ref[...]
    h = jnp.maximum(h, 0.0)
    out_ref[...] = dv * jnp.dot(h, w_ref[...], preferred_element_type=jnp.float32)


def _tc_mid(acc2, yw, dinv, bp, wp):
    din, dout = wp.shape
    return pl.pallas_call(
        _mid_body,
        grid=(N // RB,),
        in_specs=[
            pl.BlockSpec((1, RB, din), lambda i: (0, i, 0)),
            pl.BlockSpec((1, RB, din), lambda i: (1, i, 0)),
            pl.BlockSpec((RB, din), lambda i: (i, 0)),
            pl.BlockSpec((RB, 1), lambda i: (i, 0)),
            pl.BlockSpec((1, din), lambda i: (0, 0)),
            pl.BlockSpec((din, dout), lambda i: (0, 0)),
        ],
        out_specs=pl.BlockSpec((RB, dout), lambda i: (i, 0)),
        out_shape=jax.ShapeDtypeStruct((N, dout), jnp.float32),
    )(acc2, acc2, yw, dinv, bp, wp)


def _post_body(acca_ref, accb_ref, yw_ref, dinv_ref, b_ref, out_ref):
    dv = dinv_ref[...]
    out_ref[...] = dv * (acca_ref[0] + accb_ref[0] + yw_ref[...]) + b_ref[...]


def _tc_post(acc2, yw, dinv, bp):
    din = yw.shape[1]
    return pl.pallas_call(
        _post_body,
        grid=(N // RB,),
        in_specs=[
            pl.BlockSpec((1, RB, din), lambda i: (0, i, 0)),
            pl.BlockSpec((1, RB, din), lambda i: (1, i, 0)),
            pl.BlockSpec((RB, din), lambda i: (i, 0)),
            pl.BlockSpec((RB, 1), lambda i: (i, 0)),
            pl.BlockSpec((1, din), lambda i: (0, 0)),
        ],
        out_specs=pl.BlockSpec((RB, din), lambda i: (i, 0)),
        out_shape=jax.ShapeDtypeStruct((N, din), jnp.float32),
    )(acc2, acc2, yw, dinv, bp)


def _pad2(a, rows, cols):
    return jnp.pad(a, ((0, rows - a.shape[0]), (0, cols - a.shape[1])))


def _split_edges(v, kb, nb0, nb1):
    nbm = max(nb0, nb1)
    e0 = NS * nb0 * kb
    p0 = v[:e0].reshape(NS, nb0, kb)
    p1 = v[e0:].reshape(NS, nb1, kb)
    p0 = jnp.pad(p0, ((0, 0), (0, nbm - nb0), (0, 0)))
    p1 = jnp.pad(p1, ((0, 0), (0, nbm - nb1), (0, 0)))
    return jnp.stack([p0, p1])


def kernel(x, edge_index, W1, b1, W2, b2, W3, b3, W4, b4):
    f32 = jnp.float32
    src = edge_index[0].astype(jnp.int32)
    dst = edge_index[1].astype(jnp.int32)
    pad = EP - E
    srcp = jnp.concatenate([src, jnp.zeros((pad,), jnp.int32)])
    dstp = jnp.concatenate([dst, jnp.full((pad,), DUMP, jnp.int32)])
    src_r = srcp.reshape(NC, NS, NB, KB)
    dst_r = dstp.reshape(NC, NS, NB, KB)

    w1p = _pad2(W1, 256, P1)
    w2p = _pad2(W2, P1, P2)
    w3p = _pad2(W3, P2, P3)
    w4p = _pad2(W4, P3, P4)
    b1p = jnp.pad(b1, (0, P1 - b1.shape[0])).reshape(1, P1)
    b2p = jnp.pad(b2, (0, P2 - b2.shape[0])).reshape(1, P2)
    b3p = jnp.pad(b3, (0, P3 - b3.shape[0])).reshape(1, P3)
    b4p = jnp.pad(b4, (0, P4 - b4.shape[0])).reshape(1, P4)

    deg2 = _deg_kernel(dst_r)
    degt = deg2.T[:N]

    yw1, dinv = _tc_pre(x, w1p, degt)
    s64 = _split_edges(srcp, 64, 32, 128)
    d64 = _split_edges(dstp, 64, 32, 128)
    s128 = _split_edges(srcp, 128, 16, 64)
    d128 = _split_edges(dstp, 128, 16, 64)
    acc1 = _make_agg(P1, 4, 64, 32, 128)(yw1, s64, d64, jnp.zeros((NROW, P1), f32))
    yw2 = _tc_mid(acc1, yw1, dinv, b1p, w2p)
    acc2 = _make_agg(P2, 8, 128, 16, 64)(yw2, s128, d128, jnp.zeros((NROW, P2), f32))
    yw3 = _tc_mid(acc2, yw2, dinv, b2p, w3p)
    acc3 = _make_agg(P3, 8, 128, 16, 64)(yw3, s128, d128, jnp.zeros((NROW, P3), f32))
    yw4 = _tc_mid(acc3, yw3, dinv, b3p, w4p)
    acc4 = _make_agg(P4, 8, 128, 16, 64)(yw4, s128, d128, jnp.zeros((NROW, P4), f32))
    out = _tc_post(acc4, yw4, dinv, b4p)
    return out[:, :1]


# speedup vs baseline: 12.7556x; 1.0504x over previous
import functools

import jax
import jax.numpy as jnp
from jax import lax
from jax.experimental import pallas as pl
from jax.experimental.pallas import tpu as pltpu
from jax.experimental.pallas import tpu_sc as plsc

N = 10000
E = 160000
NC, NS = 2, 16
NW = NC * NS
KB = 128
NB = 40
EW = KB * NB
EP = EW * NW
NROW = 10240
RPT = NROW // NS
DUMP = N

P1, P2, P3, P4 = 112, 64, 32, 16
RB = 1000


def _deg_body(dst_hbm, deg_out, dst_v, degtab, sumv, shared):
    cid = lax.axis_index("c")
    sid = lax.axis_index("s")
    pltpu.sync_copy(dst_hbm.at[cid, sid], dst_v)

    def _zero(i, _):
        degtab[pl.ds(i * 16, 16)] = jnp.zeros((16,), jnp.float32)
        return 0
    lax.fori_loop(0, NROW // 16, _zero, 0)

    ones = jnp.ones((16,), jnp.float32)

    def _hist(i, _):
        j = i // (KB // 16)
        c = i % (KB // 16)
        idx = dst_v[j, pl.ds(c * 16, 16)]
        plsc.addupdate_scatter(degtab, [idx], ones)
        return 0
    lax.fori_loop(0, EW // 16, _hist, 0)

    pltpu.sync_copy(degtab, shared.at[sid])
    plsc.subcore_barrier()
    pltpu.sync_copy(shared.at[:, pl.ds(sid * RPT, RPT)], sumv)

    def _red(ci, _):
        a = jnp.zeros((16,), jnp.float32)
        for r in range(NS):
            a = a + sumv[r, pl.ds(ci * 16, 16)]
        degtab[pl.ds(ci * 16, 16)] = a
        return 0
    lax.fori_loop(0, RPT // 16, _red, 0)
    pltpu.sync_copy(degtab.at[pl.ds(0, RPT)], deg_out.at[cid, pl.ds(sid * RPT, RPT)])


_deg_kernel = pl.kernel(
    _deg_body,
    out_type=jax.ShapeDtypeStruct((NC, NROW), jnp.float32),
    mesh=plsc.VectorSubcoreMesh(core_axis_name="c", subcore_axis_name="s"),
    compiler_params=pltpu.CompilerParams(needs_layout_passes=False),
    scratch_types=[
        pltpu.VMEM((NB, KB), jnp.int32),
        pltpu.VMEM((NROW,), jnp.float32),
        pltpu.VMEM((NS, RPT), jnp.float32),
        pltpu.VMEM_SHARED((NS, NROW), jnp.float32),
    ],
)


def _make_agg_body(nbuf, kb, nb0, nb1):
    def _agg_body(yw_hbm, src_hbm, dst_hbm, zeros_hbm, acc_out,
                  src_v, dst_v, *rest):
        rows = rest[:nbuf]
        gsems = rest[nbuf:2 * nbuf]
        ssems = rest[2 * nbuf:3 * nbuf]
        acc_sp = rest[3 * nbuf]
        cid = lax.axis_index("c")
        sid = lax.axis_index("s")
        pltpu.sync_copy(zeros_hbm.at[pl.ds(sid * RPT, RPT)],
                        acc_sp.at[pl.ds(sid * RPT, RPT)])
        plsc.subcore_barrier()
        pltpu.sync_copy(src_hbm.at[cid, sid], src_v)
        pltpu.sync_copy(dst_hbm.at[cid, sid], dst_v)

        def _gather(i, b, sem):
            return pltpu.make_async_copy(yw_hbm.at[src_v.at[i]], rows[b], sem)

        def _scatter(i, b, sem):
            return pltpu.make_async_copy(rows[b], acc_sp.at[dst_v.at[i]], sem)

        for b in range(nbuf):
            _gather(b, b, gsems[b]).start()

        nw = jnp.where(cid == 0, nb0 // nbuf, nb1 // nbuf)

        def _wave(w, _):
            i0 = w * nbuf
            for b in range(nbuf):
                _gather(i0 + b, b, gsems[b]).wait()
                _scatter(i0 + b, b, ssems[b]).start(add=True)
            for b in range(nbuf):
                _scatter(i0 + b, b, ssems[b]).wait()
                _gather(i0 + nbuf + b, b, gsems[b]).start()
            return 0
        lax.fori_loop(0, nw - 1, _wave, 0)
        i0 = (nw - 1) * nbuf
        for b in range(nbuf):
            _gather(i0 + b, b, gsems[b]).wait()
            _scatter(i0 + b, b, ssems[b]).start(add=True)
        for b in range(nbuf):
            _scatter(i0 + b, b, ssems[b]).wait()
        plsc.subcore_barrier()
        pltpu.sync_copy(acc_sp.at[pl.ds(sid * RPT, RPT)],
                        acc_out.at[cid, pl.ds(sid * RPT, RPT)])
    return _agg_body


def _make_agg(d, nbuf, kb, nb0, nb1):
    nb = max(nb0, nb1)
    return pl.kernel(
        _make_agg_body(nbuf, kb, nb0, nb1),
        out_type=jax.ShapeDtypeStruct((NC, NROW, d), jnp.float32),
        mesh=plsc.VectorSubcoreMesh(core_axis_name="c", subcore_axis_name="s"),
        compiler_params=pltpu.CompilerParams(use_tc_tiling_on_sc=False),
        scratch_types=(
            [pltpu.VMEM((nb, kb), jnp.int32),
             pltpu.VMEM((nb, kb), jnp.int32)]
            + [pltpu.VMEM((kb, d), jnp.float32) for _ in range(nbuf)]
            + [pltpu.SemaphoreType.DMA for _ in range(2 * nbuf)]
            + [pltpu.VMEM_SHARED((NROW, d), jnp.float32)]
        ),
    )


def _pre_body(x_ref, w_ref, degt_ref, yw_ref, dinv_ref):
    d = degt_ref[:, 0:1] + degt_ref[:, 1:2] + 1.0
    dv = lax.rsqrt(d)
    xw = jnp.dot(x_ref[...], w_ref[...], preferred_element_type=jnp.float32)
    yw_ref[...] = dv * xw
    dinv_ref[...] = dv


def _tc_pre(x, w1p, degt):
    return pl.pallas_call(
        _pre_body,
        grid=(N // RB,),
        in_specs=[
            pl.BlockSpec((RB, x.shape[1]), lambda i: (i, 0)),
            pl.BlockSpec((w1p.shape[0], w1p.shape[1]), lambda i: (0, 0)),
            pl.BlockSpec((RB, 2), lambda i: (i, 0)),
        ],
        out_specs=[
            pl.BlockSpec((RB, w1p.shape[1]), lambda i: (i, 0)),
            pl.BlockSpec((RB, 1), lambda i: (i, 0)),
        ],
        out_shape=[
            jax.ShapeDtypeStruct((N, w1p.shape[1]), jnp.float32),
            jax.ShapeDtypeStruct((N, 1), jnp.float32),
        ],
    )(x, w1p, degt)


def _mid_body(acca_ref, accb_ref, yw_ref, dinv_ref, b_ref, w_ref, out_ref):
    dv = dinv_ref[...]
    h = dv * (acca_ref[0] + accb_ref[0] + yw_ref[...]) + b_ref[...]
    h = jnp.maximum(h, 0.0)
    out_ref[...] = dv * jnp.dot(h, w_ref[...], preferred_element_type=jnp.float32)


def _tc_mid(acc2, yw, dinv, bp, wp):
    din, dout = wp.shape
    return pl.pallas_call(
        _mid_body,
        grid=(N // RB,),
        in_specs=[
            pl.BlockSpec((1, RB, din), lambda i: (0, i, 0)),
            pl.BlockSpec((1, RB, din), lambda i: (1, i, 0)),
            pl.BlockSpec((RB, din), lambda i: (i, 0)),
            pl.BlockSpec((RB, 1), lambda i: (i, 0)),
            pl.BlockSpec((1, din), lambda i: (0, 0)),
            pl.BlockSpec((din, dout), lambda i: (0, 0)),
        ],
        out_specs=pl.BlockSpec((RB, dout), lambda i: (i, 0)),
        out_shape=jax.ShapeDtypeStruct((N, dout), jnp.float32),
    )(acc2, acc2, yw, dinv, bp, wp)


def _post_body(acca_ref, accb_ref, yw_ref, dinv_ref, b_ref, out_ref):
    dv = dinv_ref[...]
    out_ref[...] = dv * (acca_ref[0] + accb_ref[0] + yw_ref[...]) + b_ref[...]


def _tc_post(acc2, yw, dinv, bp):
    din = yw.shape[1]
    return pl.pallas_call(
        _post_body,
        grid=(N // RB,),
        in_specs=[
            pl.BlockSpec((1, RB, din), lambda i: (0, i, 0)),
            pl.BlockSpec((1, RB, din), lambda i: (1, i, 0)),
            pl.BlockSpec((RB, din), lambda i: (i, 0)),
            pl.BlockSpec((RB, 1), lambda i: (i, 0)),
            pl.BlockSpec((1, din), lambda i: (0, 0)),
        ],
        out_specs=pl.BlockSpec((RB, din), lambda i: (i, 0)),
        out_shape=jax.ShapeDtypeStruct((N, din), jnp.float32),
    )(acc2, acc2, yw, dinv, bp)


def _pad2(a, rows, cols):
    return jnp.pad(a, ((0, rows - a.shape[0]), (0, cols - a.shape[1])))


def _split_edges(v, kb, nb0, nb1):
    nbm = max(nb0, nb1)
    e0 = NS * nb0 * kb
    p0 = v[:e0].reshape(NS, nb0, kb)
    p1 = v[e0:].reshape(NS, nb1, kb)
    p0 = jnp.pad(p0, ((0, 0), (0, nbm - nb0), (0, 0)))
    p1 = jnp.pad(p1, ((0, 0), (0, nbm - nb1), (0, 0)))
    return jnp.stack([p0, p1])


def kernel(x, edge_index, W1, b1, W2, b2, W3, b3, W4, b4):
    f32 = jnp.float32
    src = edge_index[0].astype(jnp.int32)
    dst = edge_index[1].astype(jnp.int32)
    pad = EP - E
    srcp = jnp.concatenate([src, jnp.zeros((pad,), jnp.int32)])
    dstp = jnp.concatenate([dst, jnp.full((pad,), DUMP, jnp.int32)])
    src_r = srcp.reshape(NC, NS, NB, KB)
    dst_r = dstp.reshape(NC, NS, NB, KB)

    w1p = _pad2(W1, 256, P1)
    w2p = _pad2(W2, P1, P2)
    w3p = _pad2(W3, P2, P3)
    w4p = _pad2(W4, P3, P4)
    b1p = jnp.pad(b1, (0, P1 - b1.shape[0])).reshape(1, P1)
    b2p = jnp.pad(b2, (0, P2 - b2.shape[0])).reshape(1, P2)
    b3p = jnp.pad(b3, (0, P3 - b3.shape[0])).reshape(1, P3)
    b4p = jnp.pad(b4, (0, P4 - b4.shape[0])).reshape(1, P4)

    deg2 = _deg_kernel(dst_r)
    degt = deg2.T[:N]

    yw1, dinv = _tc_pre(x, w1p, degt)
    s64 = _split_edges(srcp, 64, 128, 32)
    d64 = _split_edges(dstp, 64, 128, 32)
    s128 = _split_edges(srcp, 128, 64, 16)
    d128 = _split_edges(dstp, 128, 64, 16)
    acc1 = _make_agg(P1, 4, 64, 128, 32)(yw1, s64, d64, jnp.zeros((NROW, P1), f32))
    yw2 = _tc_mid(acc1, yw1, dinv, b1p, w2p)
    acc2 = _make_agg(P2, 8, 128, 64, 16)(yw2, s128, d128, jnp.zeros((NROW, P2), f32))
    yw3 = _tc_mid(acc2, yw2, dinv, b2p, w3p)
    acc3 = _make_agg(P3, 8, 128, 64, 16)(yw3, s128, d128, jnp.zeros((NROW, P3), f32))
    yw4 = _tc_mid(acc3, yw3, dinv, b3p, w4p)
    acc4 = _make_agg(P4, 8, 128, 64, 16)(yw4, s128, d128, jnp.zeros((NROW, P4), f32))
    out = _tc_post(acc4, yw4, dinv, b4p)
    return out[:, :1]
